# baseline (speedup 1.0000x reference)
import jax
import jax.numpy as jnp
from jax.experimental import pallas as pl
from jax.experimental.pallas import tpu as pltpu

_INPUT = 2048
_HIDDEN = 16384
_K = 32
_B = 32
_BLK = 1024
_NBLK = _HIDDEN // _BLK
_NSLOT = 16
_OBLK = 128
_NOUT = _INPUT // _OBLK


def _body(x_ref, we_ref, b_ref, wd_ref, o_ref,
          h_s, cv_s, cg_s, t_ref, it_ref, hw_ref):
    i = pl.program_id(0)

    @pl.when(i < _NBLK)
    def _encode():
        acc = jax.lax.dot_general(
            x_ref[...], we_ref[...],
            (((1,), (1,)), ((), ())),
            preferred_element_type=jnp.float32,
        )
        blk = jnp.maximum(acc + b_ref[:, pl.ds(i * _BLK, _BLK)], 0.0)
        h_s[:, pl.ds(i * _BLK, _BLK)] = blk

        iota = jax.lax.broadcasted_iota(jnp.int32, (_B, _BLK), 1)
        work = blk
        cvs = []
        cgs = []
        for _ in range(_NSLOT):
            m = jnp.max(work, axis=1, keepdims=True)
            im = jnp.min(jnp.where(work == m, iota, _BLK),
                         axis=1, keepdims=True)
            work = jnp.where(iota == im, -1.0, work)
            cvs.append(m)
            cgs.append(im + i * _BLK)
        cv = jnp.concatenate(cvs, axis=1)
        cg = jnp.concatenate(cgs, axis=1)
        cv_s[pl.ds(i, 1), :, :] = cv[None]
        cg_s[pl.ds(i, 1), :, :] = cg[None]

    @pl.when(i == _NBLK)
    def _topk():
        def cbody(j, carry):
            Cv, Cg = carry
            m = jnp.max(Cv, axis=(0, 2), keepdims=True)
            gi = jnp.min(jnp.where(Cv == m, Cg, _HIDDEN),
                         axis=(0, 2), keepdims=True)
            Cv = jnp.where((Cv == m) & (Cg == gi), -1.0, Cv)
            t_ref[...] = m.reshape(_B, 1)
            it_ref[...] = gi.reshape(_B, 1)
            return (Cv, Cg)

        jax.lax.fori_loop(0, _K, cbody, (cv_s[...], cg_s[...]))

        h = h_s[...]
        iota = jax.lax.broadcasted_iota(jnp.int32, (_B, _HIDDEN), 1)
        keep = (h > t_ref[...]) | ((h == t_ref[...]) & (iota <= it_ref[...]))
        cnt = jnp.sum(keep.astype(jnp.int32), axis=1)
        bad = jnp.any(cnt != _K)

        @pl.when(bad)
        def _fallback():
            hw_ref[...] = h_s[...]

            def body(j, carry):
                hw = hw_ref[...]
                m = jnp.max(hw, axis=1, keepdims=True)
                im = jnp.min(jnp.where(hw == m, iota, _HIDDEN),
                             axis=1, keepdims=True)
                hw_ref[...] = jnp.where(iota == im, -1.0, hw)
                t_ref[...] = m
                it_ref[...] = im
                return carry

            jax.lax.fori_loop(0, _K, body, 0)

        keep = (h > t_ref[...]) | ((h == t_ref[...]) & (iota <= it_ref[...]))
        hw_ref[...] = jnp.where(keep, h, 0.0)

    @pl.when(i >= _NBLK)
    def _decode():
        o_ref[...] = jax.lax.dot_general(
            hw_ref[...], wd_ref[...],
            (((1,), (1,)), ((), ())),
            preferred_element_type=jnp.float32,
        )


def kernel(x, W_enc, b_enc, W_dec):
    b2 = b_enc.reshape(1, _HIDDEN)

    recon = pl.pallas_call(
        _body,
        grid=(2 * _NBLK,),
        in_specs=[
            pl.BlockSpec((_B, _INPUT), lambda i: (0, 0)),
            pl.BlockSpec((_BLK, _INPUT),
                         lambda i: (jnp.minimum(i, _NBLK - 1), 0)),
            pl.BlockSpec((1, _HIDDEN), lambda i: (0, 0)),
            pl.BlockSpec((_OBLK, _HIDDEN),
                         lambda i: (jnp.maximum(i - _NBLK, 0), 0)),
        ],
        out_specs=pl.BlockSpec((_B, _OBLK),
                               lambda i: (0, jnp.maximum(i - _NBLK, 0))),
        out_shape=jax.ShapeDtypeStruct((_B, _INPUT), jnp.float32),
        scratch_shapes=[
            pltpu.VMEM((_B, _HIDDEN), jnp.float32),
            pltpu.VMEM((_NBLK, _B, _NSLOT), jnp.float32),
            pltpu.VMEM((_NBLK, _B, _NSLOT), jnp.int32),
            pltpu.VMEM((_B, 1), jnp.float32),
            pltpu.VMEM((_B, 1), jnp.int32),
            pltpu.VMEM((_B, _HIDDEN), jnp.float32),
        ],
    )(x, W_enc, b2, W_dec)

    return recon


# speedup vs baseline: 1.0006x; 1.0006x over previous
import jax
import jax.numpy as jnp
from jax.experimental import pallas as pl
from jax.experimental.pallas import tpu as pltpu

_INPUT = 2048
_HIDDEN = 16384
_K = 32
_B = 32
_BLK = 1024
_NBLK = _HIDDEN // _BLK
_NSLOT = 16
_OBLK = 128
_NOUT = _INPUT // _OBLK


def _body(x_ref, we_ref, b_ref, wd_ref, o_ref,
          h_s, cv_s, cg_s, t_ref, it_ref, hw_ref):
    i = pl.program_id(0)

    @pl.when(i < _NBLK)
    def _encode():
        acc = jax.lax.dot_general(
            x_ref[...], we_ref[...],
            (((1,), (1,)), ((), ())),
            preferred_element_type=jnp.float32,
        )
        blk = jnp.maximum(acc + b_ref[:, pl.ds(i * _BLK, _BLK)], 0.0)
        h_s[:, pl.ds(i * _BLK, _BLK)] = blk

        iota = jax.lax.broadcasted_iota(jnp.int32, (_B, _BLK), 1)
        work = blk
        cvs = []
        cgs = []
        for _ in range(_NSLOT):
            m = jnp.max(work, axis=1, keepdims=True)
            im = jnp.min(jnp.where(work == m, iota, _BLK),
                         axis=1, keepdims=True)
            work = jnp.where(iota == im, -1.0, work)
            cvs.append(m)
            cgs.append(im + i * _BLK)
        cv = jnp.concatenate(cvs, axis=1)
        cg = jnp.concatenate(cgs, axis=1)
        cv_s[pl.ds(i, 1), :, :] = cv[None]
        cg_s[pl.ds(i, 1), :, :] = cg[None]

    @pl.when(i == _NBLK)
    def _topk():
        def cbody(j, carry):
            Cv, Cg = carry
            m = jnp.max(Cv, axis=(0, 2), keepdims=True)
            gi = jnp.min(jnp.where(Cv == m, Cg, _HIDDEN),
                         axis=(0, 2), keepdims=True)
            Cv = jnp.where((Cv == m) & (Cg == gi), -1.0, Cv)
            t_ref[...] = m.reshape(_B, 1)
            it_ref[...] = gi.reshape(_B, 1)
            return (Cv, Cg)

        jax.lax.fori_loop(0, _K, cbody, (cv_s[...], cg_s[...]))

        h = h_s[...]
        iota = jax.lax.broadcasted_iota(jnp.int32, (_B, _HIDDEN), 1)
        keep = (h > t_ref[...]) | ((h == t_ref[...]) & (iota <= it_ref[...]))
        cnt = jnp.sum(keep.astype(jnp.int32), axis=1)
        bad = jnp.any(cnt != _K)

        @pl.when(bad)
        def _fallback():
            hw_ref[...] = h_s[...]

            def body(j, carry):
                hw = hw_ref[...]
                m = jnp.max(hw, axis=1, keepdims=True)
                im = jnp.min(jnp.where(hw == m, iota, _HIDDEN),
                             axis=1, keepdims=True)
                hw_ref[...] = jnp.where(iota == im, -1.0, hw)
                t_ref[...] = m
                it_ref[...] = im
                return carry

            jax.lax.fori_loop(0, _K, body, 0)

        keep = (h > t_ref[...]) | ((h == t_ref[...]) & (iota <= it_ref[...]))
        hw_ref[...] = jnp.where(keep, h, 0.0)

    @pl.when(i >= _NBLK)
    def _decode():
        j = i - _NBLK
        acc = jax.lax.dot_general(
            hw_ref[:, pl.ds(j * _BLK, _BLK)], wd_ref[...],
            (((1,), (1,)), ((), ())),
            preferred_element_type=jnp.float32,
        )

        @pl.when(i == _NBLK)
        def _init():
            o_ref[...] = acc

        @pl.when(i > _NBLK)
        def _acc():
            o_ref[...] += acc


def kernel(x, W_enc, b_enc, W_dec):
    b2 = b_enc.reshape(1, _HIDDEN)

    recon = pl.pallas_call(
        _body,
        grid=(2 * _NBLK,),
        in_specs=[
            pl.BlockSpec((_B, _INPUT), lambda i: (0, 0)),
            pl.BlockSpec((_BLK, _INPUT),
                         lambda i: (jnp.minimum(i, _NBLK - 1), 0)),
            pl.BlockSpec((1, _HIDDEN), lambda i: (0, 0)),
            pl.BlockSpec((_INPUT, _BLK),
                         lambda i: (0, jnp.maximum(i - _NBLK, 0))),
        ],
        out_specs=pl.BlockSpec((_B, _INPUT), lambda i: (0, 0)),
        out_shape=jax.ShapeDtypeStruct((_B, _INPUT), jnp.float32),
        scratch_shapes=[
            pltpu.VMEM((_B, _HIDDEN), jnp.float32),
            pltpu.VMEM((_NBLK, _B, _NSLOT), jnp.float32),
            pltpu.VMEM((_NBLK, _B, _NSLOT), jnp.int32),
            pltpu.VMEM((_B, 1), jnp.float32),
            pltpu.VMEM((_B, 1), jnp.int32),
            pltpu.VMEM((_B, _HIDDEN), jnp.float32),
        ],
    )(x, W_enc, b2, W_dec)

    return recon


# speedup vs baseline: 1.4262x; 1.4254x over previous
import jax
import jax.numpy as jnp
from jax.experimental import pallas as pl
from jax.experimental.pallas import tpu as pltpu

_INPUT = 2048
_HIDDEN = 16384
_K = 32
_B = 32
_BLK = 1024
_NBLK = _HIDDEN // _BLK
_LT = 128
_NT = _HIDDEN // _LT
_TPB = _BLK // _LT
_NSLOT = 8


def _body(x_ref, we_ref, b_ref, wd_ref, o_ref, h_s, hs_k, t_ref, it_ref):
    i = pl.program_id(0)

    @pl.when(i < _NBLK)
    def _encode():
        acc = jax.lax.dot_general(
            x_ref[...], we_ref[...],
            (((1,), (1,)), ((), ())),
            preferred_element_type=jnp.float32,
        )
        blk = jnp.maximum(acc + b_ref[...], 0.0)
        for k in range(_TPB):
            h_s[pl.ds(i * _TPB + k, 1)] = blk[:, k * _LT:(k + 1) * _LT][None]

    @pl.when(i == _NBLK)
    def _topk():
        hk = h_s[...]
        iota_k = jax.lax.broadcasted_iota(jnp.int32, (_NT, _B, _LT), 0)
        iota_c = jax.lax.broadcasted_iota(jnp.int32, (_NT, _B, _LT), 2)
        jglob = iota_k * _LT + iota_c

        work = hk
        cvs = []
        cgs = []
        for _ in range(_NSLOT):
            m = jnp.max(work, axis=0)
            sel = work == m[None]
            km = jnp.min(jnp.where(sel, iota_k, _NT), axis=0)
            work = jnp.where(sel & (iota_k == km[None]), -1.0, work)
            cvs.append(m)
            cgs.append(km * _LT + iota_c[0])
        C = jnp.stack(cvs, axis=0)
        G = jnp.stack(cgs, axis=0)

        def cbody(j, carry):
            Cv, Cg = carry
            m = jnp.max(Cv, axis=(0, 2), keepdims=True)
            gi = jnp.min(jnp.where(Cv == m, Cg, _HIDDEN),
                         axis=(0, 2), keepdims=True)
            Cv = jnp.where((Cv == m) & (Cg == gi), -1.0, Cv)
            t_ref[...] = m.reshape(_B, 1)
            it_ref[...] = gi.reshape(_B, 1)
            return (Cv, Cg)

        jax.lax.fori_loop(0, _K, cbody, (C, G))

        tb = t_ref[...][None]
        ib = it_ref[...][None]
        keep = (hk > tb) | ((hk == tb) & (jglob <= ib))
        cnt = jnp.sum(keep.astype(jnp.int32), axis=(0, 2))
        bad = jnp.any(cnt != _K)

        @pl.when(bad)
        def _fallback():
            hs_k[...] = hk

            def body(j, carry):
                hw = hs_k[...]
                m = jnp.max(hw, axis=(0, 2), keepdims=True)
                im = jnp.min(jnp.where(hw == m, jglob, _HIDDEN),
                             axis=(0, 2), keepdims=True)
                hs_k[...] = jnp.where(jglob == im, -1.0, hw)
                t_ref[...] = m.reshape(_B, 1)
                it_ref[...] = im.reshape(_B, 1)
                return carry

            jax.lax.fori_loop(0, _K, body, 0)

        tb = t_ref[...][None]
        ib = it_ref[...][None]
        keep = (hk > tb) | ((hk == tb) & (jglob <= ib))
        hs_k[...] = jnp.where(keep, hk, 0.0)

    @pl.when(i >= _NBLK)
    def _decode():
        j = i - _NBLK
        parts = [hs_k[pl.ds(j * _TPB + k, 1)].reshape(_B, _LT)
                 for k in range(_TPB)]
        hs = jnp.concatenate(parts, axis=1)
        acc = jax.lax.dot_general(
            hs, wd_ref[...],
            (((1,), (1,)), ((), ())),
            preferred_element_type=jnp.float32,
        )

        @pl.when(i == _NBLK)
        def _init():
            o_ref[...] = acc

        @pl.when(i > _NBLK)
        def _acc():
            o_ref[...] += acc


def kernel(x, W_enc, b_enc, W_dec):
    b2 = b_enc.reshape(1, _HIDDEN)

    recon = pl.pallas_call(
        _body,
        grid=(2 * _NBLK,),
        in_specs=[
            pl.BlockSpec((_B, _INPUT), lambda i: (0, 0)),
            pl.BlockSpec((_BLK, _INPUT),
                         lambda i: (jnp.minimum(i, _NBLK - 1), 0)),
            pl.BlockSpec((1, _BLK),
                         lambda i: (0, jnp.minimum(i, _NBLK - 1))),
            pl.BlockSpec((_INPUT, _BLK),
                         lambda i: (0, jnp.maximum(i - _NBLK, 0))),
        ],
        out_specs=pl.BlockSpec((_B, _INPUT), lambda i: (0, 0)),
        out_shape=jax.ShapeDtypeStruct((_B, _INPUT), jnp.float32),
        scratch_shapes=[
            pltpu.VMEM((_NT, _B, _LT), jnp.float32),
            pltpu.VMEM((_NT, _B, _LT), jnp.float32),
            pltpu.VMEM((_B, 1), jnp.float32),
            pltpu.VMEM((_B, 1), jnp.int32),
        ],
    )(x, W_enc, b2, W_dec)

    return recon


# speedup vs baseline: 1.4504x; 1.0170x over previous
import jax
import jax.numpy as jnp
from jax.experimental import pallas as pl
from jax.experimental.pallas import tpu as pltpu

_INPUT = 2048
_HIDDEN = 16384
_K = 32
_B = 32
_BLK = 1024
_NBLK = _HIDDEN // _BLK
_LT = 128
_NT = _HIDDEN // _LT
_TPB = _BLK // _LT
_NSLOT = 8


def _body(x_ref, wea_ref, web_ref, b_ref, wda_ref, wdb_ref, o_ref,
          h_s, hs_k, t_ref, it_ref):
    i = pl.program_id(0)

    @pl.when(i < _NBLK)
    def _encode():
        acc_a = jax.lax.dot_general(
            x_ref[...], wea_ref[...],
            (((1,), (1,)), ((), ())),
            preferred_element_type=jnp.float32,
        )
        acc_b = jax.lax.dot_general(
            x_ref[...], web_ref[...],
            (((1,), (1,)), ((), ())),
            preferred_element_type=jnp.float32,
        )
        acc = jnp.concatenate([acc_a, acc_b], axis=1)
        blk = jnp.maximum(acc + b_ref[...], 0.0)
        for k in range(_TPB):
            h_s[pl.ds(i * _TPB + k, 1)] = blk[:, k * _LT:(k + 1) * _LT][None]

    @pl.when(i == _NBLK)
    def _topk():
        hk = h_s[...]
        iota_k = jax.lax.broadcasted_iota(jnp.int32, (_NT, _B, _LT), 0)
        iota_c = jax.lax.broadcasted_iota(jnp.int32, (_NT, _B, _LT), 2)
        jglob = iota_k * _LT + iota_c

        work = hk
        cvs = []
        cgs = []
        for _ in range(_NSLOT):
            m = jnp.max(work, axis=0)
            sel = work == m[None]
            km = jnp.min(jnp.where(sel, iota_k, _NT), axis=0)
            work = jnp.where(sel & (iota_k == km[None]), -1.0, work)
            cvs.append(m)
            cgs.append(km * _LT + iota_c[0])
        C = jnp.stack(cvs, axis=0)
        G = jnp.stack(cgs, axis=0)

        def cbody(j, carry):
            Cv, Cg = carry
            m = jnp.max(Cv, axis=(0, 2), keepdims=True)
            gi = jnp.min(jnp.where(Cv == m, Cg, _HIDDEN),
                         axis=(0, 2), keepdims=True)
            Cv = jnp.where((Cv == m) & (Cg == gi), -1.0, Cv)
            t_ref[...] = m.reshape(_B, 1)
            it_ref[...] = gi.reshape(_B, 1)
            return (Cv, Cg)

        jax.lax.fori_loop(0, _K, cbody, (C, G))

        tb = t_ref[...][None]
        ib = it_ref[...][None]
        keep = (hk > tb) | ((hk == tb) & (jglob <= ib))
        cnt = jnp.sum(keep.astype(jnp.int32), axis=(0, 2))
        bad = jnp.any(cnt != _K)

        @pl.when(bad)
        def _fallback():
            hs_k[...] = hk

            def body(j, carry):
                hw = hs_k[...]
                m = jnp.max(hw, axis=(0, 2), keepdims=True)
                im = jnp.min(jnp.where(hw == m, jglob, _HIDDEN),
                             axis=(0, 2), keepdims=True)
                hs_k[...] = jnp.where(jglob == im, -1.0, hw)
                t_ref[...] = m.reshape(_B, 1)
                it_ref[...] = im.reshape(_B, 1)
                return carry

            jax.lax.fori_loop(0, _K, body, 0)

        tb = t_ref[...][None]
        ib = it_ref[...][None]
        keep = (hk > tb) | ((hk == tb) & (jglob <= ib))
        hs_k[...] = jnp.where(keep, hk, 0.0)

    @pl.when(i >= _NBLK)
    def _decode():
        j = i - _NBLK
        parts = [hs_k[pl.ds(j * _TPB + k, 1)].reshape(_B, _LT)
                 for k in range(_TPB)]
        hs_a = jnp.concatenate(parts[:_TPB // 2], axis=1)
        hs_b = jnp.concatenate(parts[_TPB // 2:], axis=1)
        acc = jax.lax.dot_general(
            hs_a, wda_ref[...],
            (((1,), (1,)), ((), ())),
            preferred_element_type=jnp.float32,
        ) + jax.lax.dot_general(
            hs_b, wdb_ref[...],
            (((1,), (1,)), ((), ())),
            preferred_element_type=jnp.float32,
        )

        @pl.when(i == _NBLK)
        def _init():
            o_ref[...] = acc

        @pl.when(i > _NBLK)
        def _acc():
            o_ref[...] += acc


def kernel(x, W_enc, b_enc, W_dec):
    b2 = b_enc.reshape(1, _HIDDEN)

    recon = pl.pallas_call(
        _body,
        grid=(2 * _NBLK,),
        in_specs=[
            pl.BlockSpec((_B, _INPUT), lambda i: (0, 0)),
            pl.BlockSpec((_BLK // 2, _INPUT),
                         lambda i: (2 * jnp.minimum(i, _NBLK - 1), 0)),
            pl.BlockSpec((_BLK // 2, _INPUT),
                         lambda i: (2 * jnp.minimum(i, _NBLK - 1) + 1, 0)),
            pl.BlockSpec((1, _BLK),
                         lambda i: (0, jnp.minimum(i, _NBLK - 1))),
            pl.BlockSpec((_INPUT, _BLK // 2),
                         lambda i: (0, 2 * jnp.maximum(i - _NBLK, 0))),
            pl.BlockSpec((_INPUT, _BLK // 2),
                         lambda i: (0, 2 * jnp.maximum(i - _NBLK, 0) + 1)),
        ],
        out_specs=pl.BlockSpec((_B, _INPUT), lambda i: (0, 0)),
        out_shape=jax.ShapeDtypeStruct((_B, _INPUT), jnp.float32),
        scratch_shapes=[
            pltpu.VMEM((_NT, _B, _LT), jnp.float32),
            pltpu.VMEM((_NT, _B, _LT), jnp.float32),
            pltpu.VMEM((_B, 1), jnp.float32),
            pltpu.VMEM((_B, 1), jnp.int32),
        ],
    )(x, W_enc, W_enc, b2, W_dec, W_dec)

    return recon


# speedup vs baseline: 1.4748x; 1.0168x over previous
import jax
import jax.numpy as jnp
from jax.experimental import pallas as pl
from jax.experimental.pallas import tpu as pltpu

_INPUT = 2048
_HIDDEN = 16384
_K = 32
_B = 32
_BLK = 1024
_NBLK = _HIDDEN // _BLK
_LT = 128
_NT = _HIDDEN // _LT
_TPB = _BLK // _LT
_NSLOT = 8


def _body(x_ref, we_ref, b_ref, wd_ref, o_ref,
          h_s, hs_k, cv_s, cg_s, t_ref, it_ref):
    i = pl.program_id(0)

    @pl.when(i < _NBLK)
    def _encode():
        acc = jax.lax.dot_general(
            x_ref[...], we_ref[...],
            (((1,), (1,)), ((), ())),
            preferred_element_type=jnp.float32,
        )
        blk = jnp.maximum(acc + b_ref[...], 0.0)
        for k in range(_TPB):
            h_s[pl.ds(i * _TPB + k, 1)] = blk[:, k * _LT:(k + 1) * _LT][None]

        iota_c2 = jax.lax.broadcasted_iota(jnp.int32, (_B, _LT), 1)
        iota_s = jax.lax.broadcasted_iota(jnp.int32, (_NSLOT, _B, _LT), 0)

        @pl.when(i == 0)
        def _cand_init():
            tiles = [blk[:, k * _LT:(k + 1) * _LT] for k in range(_TPB)]
            cv_s[...] = jnp.stack(tiles, axis=0)
            cg_s[...] = (iota_s * _LT + iota_c2[None])

        @pl.when(i > 0)
        def _cand_update():
            cv = cv_s[...]
            cg = cg_s[...]
            for k in range(_TPB):
                v = blk[:, k * _LT:(k + 1) * _LT]
                g = (i * _TPB + k) * _LT + iota_c2
                minv = jnp.min(cv, axis=0)
                ksl = jnp.min(jnp.where(cv == minv[None], iota_s, _NSLOT),
                              axis=0)
                sel = (iota_s == ksl[None]) & (v > minv)[None]
                cv = jnp.where(sel, v[None], cv)
                cg = jnp.where(sel, g[None], cg)
            cv_s[...] = cv
            cg_s[...] = cg

    @pl.when(i == _NBLK)
    def _topk():
        hk = h_s[...]
        iota_k = jax.lax.broadcasted_iota(jnp.int32, (_NT, _B, _LT), 0)
        iota_c = jax.lax.broadcasted_iota(jnp.int32, (_NT, _B, _LT), 2)
        jglob = iota_k * _LT + iota_c

        C = cv_s[...]
        G = cg_s[...]

        def cbody(j, carry):
            Cv, Cg = carry
            m = jnp.max(Cv, axis=(0, 2), keepdims=True)
            gi = jnp.min(jnp.where(Cv == m, Cg, _HIDDEN),
                         axis=(0, 2), keepdims=True)
            Cv = jnp.where((Cv == m) & (Cg == gi), -1.0, Cv)
            t_ref[...] = m.reshape(_B, 1)
            it_ref[...] = gi.reshape(_B, 1)
            return (Cv, Cg)

        jax.lax.fori_loop(0, _K, cbody, (C, G))

        tb = t_ref[...][None]
        ib = it_ref[...][None]
        keep = (hk > tb) | ((hk == tb) & (jglob <= ib))
        cnt = jnp.sum(keep.astype(jnp.int32), axis=(0, 2))
        bad = jnp.any(cnt != _K)

        @pl.when(bad)
        def _fallback():
            hs_k[...] = hk

            def body(j, carry):
                hw = hs_k[...]
                m = jnp.max(hw, axis=(0, 2), keepdims=True)
                im = jnp.min(jnp.where(hw == m, jglob, _HIDDEN),
                             axis=(0, 2), keepdims=True)
                hs_k[...] = jnp.where(jglob == im, -1.0, hw)
                t_ref[...] = m.reshape(_B, 1)
                it_ref[...] = im.reshape(_B, 1)
                return carry

            jax.lax.fori_loop(0, _K, body, 0)

        tb = t_ref[...][None]
        ib = it_ref[...][None]
        keep = (hk > tb) | ((hk == tb) & (jglob <= ib))
        hs_k[...] = jnp.where(keep, hk, 0.0)

    @pl.when(i >= _NBLK)
    def _decode():
        j = i - _NBLK
        parts = [hs_k[pl.ds(j * _TPB + k, 1)].reshape(_B, _LT)
                 for k in range(_TPB)]
        hs = jnp.concatenate(parts, axis=1)
        acc = jax.lax.dot_general(
            hs, wd_ref[...],
            (((1,), (1,)), ((), ())),
            preferred_element_type=jnp.float32,
        )

        @pl.when(i == _NBLK)
        def _init():
            o_ref[...] = acc

        @pl.when(i > _NBLK)
        def _acc():
            o_ref[...] += acc


def kernel(x, W_enc, b_enc, W_dec):
    b2 = b_enc.reshape(1, _HIDDEN)

    recon = pl.pallas_call(
        _body,
        grid=(2 * _NBLK,),
        in_specs=[
            pl.BlockSpec((_B, _INPUT), lambda i: (0, 0)),
            pl.BlockSpec((_BLK, _INPUT),
                         lambda i: (jnp.minimum(i, _NBLK - 1), 0)),
            pl.BlockSpec((1, _BLK),
                         lambda i: (0, jnp.minimum(i, _NBLK - 1))),
            pl.BlockSpec((_INPUT, _BLK),
                         lambda i: (0, jnp.maximum(i - _NBLK, 0))),
        ],
        out_specs=pl.BlockSpec((_B, _INPUT), lambda i: (0, 0)),
        out_shape=jax.ShapeDtypeStruct((_B, _INPUT), jnp.float32),
        scratch_shapes=[
            pltpu.VMEM((_NT, _B, _LT), jnp.float32),
            pltpu.VMEM((_NT, _B, _LT), jnp.float32),
            pltpu.VMEM((_NSLOT, _B, _LT), jnp.float32),
            pltpu.VMEM((_NSLOT, _B, _LT), jnp.int32),
            pltpu.VMEM((_B, 1), jnp.float32),
            pltpu.VMEM((_B, 1), jnp.int32),
        ],
    )(x, W_enc, b2, W_dec)

    return recon
